# baseline (speedup 1.0000x reference)
import jax
import jax.numpy as jnp
from jax.experimental import pallas as pl

N = 1000
H = 104
W = 104
BH = 8
NHB = 13
MASK_THR = 0.5
SIGMA = 2.0


def _gram_kernel(soft_ref, g_ref):
    hb = pl.program_id(0)
    x = soft_ref[...]
    b = (x > MASK_THR).astype(jnp.bfloat16)
    b2 = b.reshape(N, BH * W)
    part = jax.lax.dot_general(
        b2, b2, (((1,), (1,)), ((), ())), preferred_element_type=jnp.float32)

    @pl.when(hb == 0)
    def _():
        g_ref[...] = part

    @pl.when(hb != 0)
    def _():
        g_ref[...] += part


def _nms_kernel(g_ref, sr_ref, sc_ref, lr_ref, lc_ref, out_ref):
    g = g_ref[...]
    sr = sr_ref[...]
    sc = sc_ref[...]
    lr = lr_ref[...]
    lc = lc_ref[...]
    iu = jax.lax.broadcasted_iota(jnp.int32, (N, N), 0)
    iv = jax.lax.broadcasted_iota(jnp.int32, (N, N), 1)

    diag = iu == iv
    s_col = jnp.sum(jnp.where(diag, g, 0.0), axis=1, keepdims=True)
    s_row = jnp.sum(jnp.where(diag, g, 0.0), axis=0, keepdims=True)

    den = s_col + s_row - g
    iou = jnp.where(den > 0.0, g, 0.0) / jnp.where(den > 0.0, den, 1.0)

    order = (sc > sr) | ((sc == sr) & (iu < iv))
    ordt = (sr > sc) | ((sr == sc) & (iv < iu))
    lbl = lc == lr

    m = jnp.where(order & lbl, iou, 0.0)
    mt = jnp.where(ordt & lbl, iou, 0.0)

    c_row = jnp.max(m, axis=0, keepdims=True)
    logr = c_row * c_row - mt * mt
    d_col = jnp.exp(SIGMA * jnp.min(logr, axis=1, keepdims=True))

    val_col = sc * d_col
    rank_col = jnp.sum(ordt.astype(jnp.float32), axis=1, keepdims=True)
    onehot = rank_col == iv.astype(jnp.float32)
    out_ref[...] = jnp.sum(jnp.where(onehot, val_col, 0.0),
                           axis=0, keepdims=True)


def kernel(seg_masks_soft, cate_scores, cate_labels):
    g = pl.pallas_call(
        _gram_kernel,
        grid=(NHB,),
        in_specs=[pl.BlockSpec((N, BH, W), lambda hb: (0, hb, 0))],
        out_specs=pl.BlockSpec((N, N), lambda hb: (0, 0)),
        out_shape=jax.ShapeDtypeStruct((N, N), jnp.float32),
    )(seg_masks_soft)

    sr = cate_scores.reshape(1, N)
    sc = cate_scores.reshape(N, 1)
    lr = cate_labels.reshape(1, N)
    lc = cate_labels.reshape(N, 1)
    out = pl.pallas_call(
        _nms_kernel,
        out_shape=jax.ShapeDtypeStruct((1, N), jnp.float32),
    )(g, sr, sc, lr, lc)
    return out.reshape(N)


# speedup vs baseline: 3.3271x; 3.3271x over previous
import jax
import jax.numpy as jnp
from jax.experimental import pallas as pl

N = 1000
K = 104 * 104
BK = 2704
NKB = 4
MASK_THR = 0.5
SIGMA = 2.0


def _gram_kernel(xt_ref, g_ref):
    kb = pl.program_id(0)
    x = xt_ref[...]
    b = (x > MASK_THR).astype(jnp.float8_e4m3fn)
    part = jax.lax.dot_general(
        b, b, (((0,), (0,)), ((), ())), preferred_element_type=jnp.float32)

    @pl.when(kb == 0)
    def _():
        g_ref[...] = part

    @pl.when(kb != 0)
    def _():
        g_ref[...] += part


def _nms_kernel(g_ref, sr_ref, sc_ref, lr_ref, lc_ref, out_ref):
    g = g_ref[...]
    sr = sr_ref[...]
    sc = sc_ref[...]
    lr = lr_ref[...]
    lc = lc_ref[...]
    iu = jax.lax.broadcasted_iota(jnp.int32, (N, N), 0)
    iv = jax.lax.broadcasted_iota(jnp.int32, (N, N), 1)

    diag = iu == iv
    s_col = jnp.sum(jnp.where(diag, g, 0.0), axis=1, keepdims=True)
    s_row = jnp.sum(jnp.where(diag, g, 0.0), axis=0, keepdims=True)

    den = s_col + s_row - g
    iou = jnp.where(den > 0.0, g, 0.0) / jnp.where(den > 0.0, den, 1.0)

    order = (sc > sr) | ((sc == sr) & (iu < iv))
    ordt = (sr > sc) | ((sr == sc) & (iv < iu))
    lbl = lc == lr

    m = jnp.where(order & lbl, iou, 0.0)
    mt = jnp.where(ordt & lbl, iou, 0.0)

    c_row = jnp.max(m, axis=0, keepdims=True)
    logr = c_row * c_row - mt * mt
    d_col = jnp.exp(SIGMA * jnp.min(logr, axis=1, keepdims=True))

    val_col = sc * d_col
    rank_col = jnp.sum(ordt.astype(jnp.float32), axis=1, keepdims=True)
    onehot = rank_col == iv.astype(jnp.float32)
    out_ref[...] = jnp.sum(jnp.where(onehot, val_col, 0.0),
                           axis=0, keepdims=True)


def kernel(seg_masks_soft, cate_scores, cate_labels):
    xt = seg_masks_soft.transpose(1, 2, 0).reshape(K, N)
    g = pl.pallas_call(
        _gram_kernel,
        grid=(NKB,),
        in_specs=[pl.BlockSpec((BK, N), lambda kb: (kb, 0))],
        out_specs=pl.BlockSpec((N, N), lambda kb: (0, 0)),
        out_shape=jax.ShapeDtypeStruct((N, N), jnp.float32),
    )(xt)

    sr = cate_scores.reshape(1, N)
    sc = cate_scores.reshape(N, 1)
    lr = cate_labels.reshape(1, N)
    lc = cate_labels.reshape(N, 1)
    out = pl.pallas_call(
        _nms_kernel,
        out_shape=jax.ShapeDtypeStruct((1, N), jnp.float32),
    )(g, sr, sc, lr, lc)
    return out.reshape(N)


# speedup vs baseline: 3.8302x; 1.1512x over previous
import jax
import jax.numpy as jnp
from jax.experimental import pallas as pl
from jax.experimental.pallas import tpu as pltpu

N = 1000
K = 104 * 104
BK = 2704
NKB = 4
MASK_THR = 0.5
SIGMA = 2.0


def _nms_epilogue(g, sr, sc, lr, lc):
    iu = jax.lax.broadcasted_iota(jnp.int32, (N, N), 0)
    iv = jax.lax.broadcasted_iota(jnp.int32, (N, N), 1)

    diag = iu == iv
    s_col = jnp.sum(jnp.where(diag, g, 0.0), axis=1, keepdims=True)
    s_row = jnp.sum(jnp.where(diag, g, 0.0), axis=0, keepdims=True)

    den = s_col + s_row - g
    iou = jnp.where(den > 0.0, g, 0.0) / jnp.where(den > 0.0, den, 1.0)

    order = (sc > sr) | ((sc == sr) & (iu < iv))
    ordt = (sr > sc) | ((sr == sc) & (iv < iu))
    lbl = lc == lr

    m = jnp.where(order & lbl, iou, 0.0)
    mt = jnp.where(ordt & lbl, iou, 0.0)

    c_row = jnp.max(m, axis=0, keepdims=True)
    logr = c_row * c_row - mt * mt
    d_col = jnp.exp(SIGMA * jnp.min(logr, axis=1, keepdims=True))

    val_col = sc * d_col
    rank_col = jnp.sum(ordt.astype(jnp.float32), axis=1, keepdims=True)
    onehot = rank_col == iv.astype(jnp.float32)
    return jnp.sum(jnp.where(onehot, val_col, 0.0),
                   axis=0, keepdims=True)


def _fused_kernel(xt_ref, sr_ref, sc_ref, lr_ref, lc_ref, out_ref, g_scr):
    kb = pl.program_id(0)
    x = xt_ref[...]
    b = (x > MASK_THR).astype(jnp.float8_e4m3fn)
    part = jax.lax.dot_general(
        b, b, (((0,), (0,)), ((), ())), preferred_element_type=jnp.float32)

    @pl.when(kb == 0)
    def _():
        g_scr[...] = part

    @pl.when(kb != 0)
    def _():
        g_scr[...] += part

    @pl.when(kb == NKB - 1)
    def _():
        out_ref[...] = _nms_epilogue(
            g_scr[...], sr_ref[...], sc_ref[...], lr_ref[...], lc_ref[...])


def kernel(seg_masks_soft, cate_scores, cate_labels):
    xt = seg_masks_soft.transpose(1, 2, 0).reshape(K, N)
    sr = cate_scores.reshape(1, N)
    sc = cate_scores.reshape(N, 1)
    lr = cate_labels.reshape(1, N)
    lc = cate_labels.reshape(N, 1)
    out = pl.pallas_call(
        _fused_kernel,
        grid=(NKB,),
        in_specs=[
            pl.BlockSpec((BK, N), lambda kb: (kb, 0)),
            pl.BlockSpec((1, N), lambda kb: (0, 0)),
            pl.BlockSpec((N, 1), lambda kb: (0, 0)),
            pl.BlockSpec((1, N), lambda kb: (0, 0)),
            pl.BlockSpec((N, 1), lambda kb: (0, 0)),
        ],
        out_specs=pl.BlockSpec((1, N), lambda kb: (0, 0)),
        out_shape=jax.ShapeDtypeStruct((1, N), jnp.float32),
        scratch_shapes=[pltpu.VMEM((N, N), jnp.float32)],
    )(xt, sr, sc, lr, lc)
    return out.reshape(N)


# speedup vs baseline: 4.3636x; 1.1393x over previous
import jax
import jax.numpy as jnp
from jax.experimental import pallas as pl
from jax.experimental.pallas import tpu as pltpu

N = 1000
K = 104 * 104
BK = 2704
NKB = 4
MASK_THR = 0.5
SIGMA = 2.0


def _nms_epilogue(g, sr, lr):
    iu = jax.lax.broadcasted_iota(jnp.int32, (N, N), 0)
    iv = jax.lax.broadcasted_iota(jnp.int32, (N, N), 1)
    diag = iu == iv

    sc = jnp.sum(jnp.where(diag, jnp.broadcast_to(sr, (N, N)), 0.0),
                 axis=1, keepdims=True)
    lc = jnp.sum(jnp.where(diag, jnp.broadcast_to(lr, (N, N)), 0),
                 axis=1, keepdims=True)

    s_col = jnp.sum(jnp.where(diag, g, 0.0), axis=1, keepdims=True)
    s_row = jnp.sum(jnp.where(diag, g, 0.0), axis=0, keepdims=True)

    den = s_col + s_row - g
    iou = jnp.where(den > 0.0, g, 0.0) / jnp.where(den > 0.0, den, 1.0)

    order = (sc > sr) | ((sc == sr) & (iu < iv))
    ordt = (sr > sc) | ((sr == sc) & (iv < iu))
    lbl = lc == lr

    m = jnp.where(order & lbl, iou, 0.0)
    mt = jnp.where(ordt & lbl, iou, 0.0)

    c_row = jnp.max(m, axis=0, keepdims=True)
    logr = c_row * c_row - mt * mt
    d_col = jnp.exp(SIGMA * jnp.min(logr, axis=1, keepdims=True))

    val_col = sc * d_col
    rank_col = jnp.sum(ordt.astype(jnp.float32), axis=1, keepdims=True)
    onehot = rank_col == iv.astype(jnp.float32)
    return jnp.sum(jnp.where(onehot, val_col, 0.0),
                   axis=0, keepdims=True)


def _fused_kernel(xt_ref, sr_ref, lr_ref, out_ref, g_scr):
    kb = pl.program_id(0)
    x = xt_ref[...]
    b = (x > MASK_THR).astype(jnp.float8_e4m3fn)
    part = jax.lax.dot_general(
        b, b, (((0,), (0,)), ((), ())), preferred_element_type=jnp.float32)

    @pl.when(kb == 0)
    def _():
        g_scr[...] = part

    @pl.when(kb != 0)
    def _():
        g_scr[...] += part

    @pl.when(kb == NKB - 1)
    def _():
        out_ref[...] = _nms_epilogue(g_scr[...], sr_ref[...], lr_ref[...])


def kernel(seg_masks_soft, cate_scores, cate_labels):
    xt = seg_masks_soft.transpose(1, 2, 0).reshape(K, N)
    sr = cate_scores.reshape(1, N)
    lr = cate_labels.reshape(1, N)
    out = pl.pallas_call(
        _fused_kernel,
        grid=(NKB,),
        in_specs=[
            pl.BlockSpec((BK, N), lambda kb: (kb, 0)),
            pl.BlockSpec((1, N), lambda kb: (0, 0)),
            pl.BlockSpec((1, N), lambda kb: (0, 0)),
        ],
        out_specs=pl.BlockSpec((1, N), lambda kb: (0, 0)),
        out_shape=jax.ShapeDtypeStruct((1, N), jnp.float32),
        scratch_shapes=[pltpu.VMEM((N, N), jnp.float32)],
    )(xt, sr, lr)
    return out.reshape(N)
